# baseline (speedup 1.0000x reference)
import jax
import jax.numpy as jnp
from jax.experimental import pallas as pl

N = 10000
BM = 400


def _s1_body(x_ref, w1_ref, s1_ref):
    x = x_ref[...].astype(jnp.bfloat16)
    w = w1_ref[...].astype(jnp.bfloat16)
    s1_ref[...] = jnp.dot(x, w, preferred_element_type=jnp.float32)


def _pass1_body(adj_ref, s1_ref, b1_ref, w2_ref, s2_ref):
    a = adj_ref[...].astype(jnp.bfloat16)
    s1 = s1_ref[...].astype(jnp.bfloat16)
    h1 = jnp.dot(a, s1, preferred_element_type=jnp.float32) + b1_ref[...]
    h1r = jnp.maximum(h1, 0.0)
    s2_ref[...] = jnp.dot(h1r.astype(jnp.bfloat16),
                          w2_ref[...].astype(jnp.bfloat16),
                          preferred_element_type=jnp.float32)


def _pass2_body(adj_ref, s2_ref, b2_ref, o_ref):
    a = adj_ref[...].astype(jnp.bfloat16)
    s2 = s2_ref[...].astype(jnp.bfloat16)
    z = jnp.dot(a, s2, preferred_element_type=jnp.float32) + b2_ref[...]
    m = jnp.max(z, axis=1, keepdims=True)
    e = jnp.exp(z - m)
    lse = jnp.log(jnp.sum(e, axis=1, keepdims=True)) + m
    o_ref[...] = z - lse


def kernel(x, adj, W1, b1, W2, b2):
    nfeat = x.shape[1]
    nhid = W1.shape[1]
    nclass = W2.shape[1]
    b1r = b1.reshape(1, nhid)
    b2r = b2.reshape(1, nclass)

    s1 = pl.pallas_call(
        _s1_body,
        out_shape=jax.ShapeDtypeStruct((N, nhid), jnp.float32),
    )(x, W1)

    grid = (N // BM,)
    s2 = pl.pallas_call(
        _pass1_body,
        grid=grid,
        in_specs=[
            pl.BlockSpec((BM, N), lambda i: (i, 0)),
            pl.BlockSpec((N, nhid), lambda i: (0, 0)),
            pl.BlockSpec((1, nhid), lambda i: (0, 0)),
            pl.BlockSpec((nhid, nclass), lambda i: (0, 0)),
        ],
        out_specs=pl.BlockSpec((BM, nclass), lambda i: (i, 0)),
        out_shape=jax.ShapeDtypeStruct((N, nclass), jnp.float32),
    )(adj, s1, b1r, W2)

    out = pl.pallas_call(
        _pass2_body,
        grid=grid,
        in_specs=[
            pl.BlockSpec((BM, N), lambda i: (i, 0)),
            pl.BlockSpec((N, nclass), lambda i: (0, 0)),
            pl.BlockSpec((1, nclass), lambda i: (0, 0)),
        ],
        out_specs=pl.BlockSpec((BM, nclass), lambda i: (i, 0)),
        out_shape=jax.ShapeDtypeStruct((N, nclass), jnp.float32),
    )(adj, s2, b2r)

    return out
